# baseline (speedup 1.0000x reference)
import functools

import jax
import jax.numpy as jnp
import numpy as np
from jax import lax
from jax.experimental import pallas as pl
from jax.experimental.pallas import tpu as pltpu
from jax.experimental.pallas import tpu_sc as plsc

N = 10000
E = 160000
H = 256
R = 6
K = 8
NH = 4
HD = H // NH
BN = 1000
GRID = N // BN
_INV_SQRT_HD = 1.0 / np.sqrt(HD).astype(np.float32)


def _ln(x, g, b):
    mu = jnp.mean(x, axis=-1, keepdims=True)
    xc = x - mu
    var = jnp.mean(xc * xc, axis=-1, keepdims=True)
    return xc * jax.lax.rsqrt(var + 1e-5) * g + b


def _dot(a, b):
    return jnp.dot(a, b, preferred_element_type=jnp.float32)


def _stage_a_body(x_r, nt_r, pos_r, wtT, bt, gt, bet, waT, ba, ga, bea,
                  ntemb, w1T, b1, w2T, b2, o_r):
    xb = x_r[...]
    ht = jnp.maximum(_ln(_dot(xb, wtT[...]) + bt[...], gt[...], bet[...]), 0.0)
    ha = jnp.maximum(_ln(_dot(xb, waT[...]) + ba[...], ga[...], bea[...]), 0.0)
    m0 = nt_r[...] == 0
    h = jnp.where(m0, ht, ha)
    h = h + jnp.where(m0, ntemb[0:1, :], ntemb[1:2, :])
    pe = _dot(jnp.maximum(_dot(pos_r[...], w1T[...]) + b1[...], 0.0),
              w2T[...]) + b2[...]
    o_r[...] = h + pe


def _stage_a(x, nt2, pos, p):
    full = lambda *s: pl.BlockSpec(s, lambda i: (0,) * len(s))
    return pl.pallas_call(
        _stage_a_body,
        grid=(GRID,),
        in_specs=[
            pl.BlockSpec((BN, 777), lambda i: (i, 0)),
            pl.BlockSpec((BN, 1), lambda i: (i, 0)),
            pl.BlockSpec((BN, 1), lambda i: (i, 0)),
            full(777, H), full(1, H), full(1, H), full(1, H),
            full(777, H), full(1, H), full(1, H), full(1, H),
            full(2, H),
            full(1, H // 2), full(1, H // 2), full(H // 2, H), full(1, H),
        ],
        out_specs=pl.BlockSpec((BN, H), lambda i: (i, 0)),
        out_shape=jax.ShapeDtypeStruct((N, H), jnp.float32),
    )(x, nt2, pos,
      p['text_W'].T, p['text_b'][None], p['text_g'][None], p['text_be'][None],
      p['aud_W'].T, p['aud_b'][None], p['aud_g'][None], p['aud_be'][None],
      p['nt_emb'],
      p['pos_W1'].T, p['pos_b1'][None], p['pos_W2'].T, p['pos_b2'][None])


def _qkv_body(h_r, wqT, wkT, wvT, q_r, kv_r):
    hb = h_r[...]
    for r in range(R):
        q_r[r] = _dot(hb, wqT[r])
        kv_r[r, :, 0:H] = _dot(hb, wkT[r])
        kv_r[r, :, H:2 * H] = _dot(hb, wvT[r])


def _qkv(h, lp):
    full3 = pl.BlockSpec((R, H, H), lambda i: (0, 0, 0))
    return pl.pallas_call(
        _qkv_body,
        grid=(GRID,),
        in_specs=[pl.BlockSpec((BN, H), lambda i: (i, 0)), full3, full3, full3],
        out_specs=[pl.BlockSpec((R, BN, H), lambda i: (0, i, 0)),
                   pl.BlockSpec((R, BN, 2 * H), lambda i: (0, i, 0))],
        out_shape=[jax.ShapeDtypeStruct((R, N, H), jnp.float32),
                   jax.ShapeDtypeStruct((R, N, 2 * H), jnp.float32)],
    )(h, jnp.transpose(lp['Wq'], (0, 2, 1)),
      jnp.transpose(lp['Wk'], (0, 2, 1)),
      jnp.transpose(lp['Wv'], (0, 2, 1)))


SEG_ROWS = 61440
OUT_ROWS = SEG_ROWS + 32
DP = 128


_GDN = lax.GatherDimensionNumbers(offset_dims=(), collapsed_slice_dims=(0,),
                                  start_index_map=(0,))


def _perm(v, idx):
    return lax.gather(v, idx[:, None], _GDN, (1,),
                      mode=lax.GatherScatterMode.PROMISE_IN_BOUNDS)


def _allsum(v, iota):
    for step in (1, 2, 4, 8):
        v = v + _perm(v, jnp.bitwise_xor(iota, step))
    return v


def _edge_body(qt, kvt, eidx, cuts, zb,
               aggu_out, den_out,
               qba, kvba, stga, dsta, eia,
               qbb, kvbb, stgb, dstb, eib,
               ct_v, zrows,
               sem_ga, sem_gb, sem_ia, sem_ib, sem_sa, sem_sb):
    c = lax.axis_index("c")
    s = lax.axis_index("s")
    w = c * 16 + s
    iota = lax.iota(jnp.int32, 16)
    pltpu.sync_copy(zb, zrows)
    pltpu.sync_copy(cuts, ct_v)
    for dd in (dsta, dstb):
        for e in range(16):
            for j in range(1, DP // 16):
                dd[e, pl.ds(j * 16, 16)] = jnp.zeros((16,), jnp.float32)

    def ext(off):
        a0 = ct_v[pl.ds(off, 16)]
        a1 = ct_v[pl.ds(off + 16, 16)]
        ga = jnp.clip(w, 0, 15)
        gb = jnp.clip(w - 16, 0, 15)
        av = _perm(a0, jnp.where(iota == 0, jnp.zeros_like(iota) + ga, iota))[0]
        bv = _perm(a1, jnp.where(iota == 0, jnp.zeros_like(iota) + gb, iota))[0]
        return jnp.where(w < 16, av, bv)

    e_lo = ext(0)
    e_hi = ext(32)
    s_lo = ext(64)
    s_hi = ext(96)

    nz = (s_hi - s_lo + 15) // 16

    def zero_body(j, carry):
        idxz = jnp.minimum(s_lo + j * 16 + iota, s_hi - 1)
        z1 = pltpu.async_copy(zrows, aggu_out.at[idxz], sem_sa)
        z2 = pltpu.async_copy(zrows.at[pl.ds(0, 16), pl.ds(0, DP)],
                              den_out.at[idxz], sem_sa)
        z1.wait()
        z2.wait()
        return carry

    lax.fori_loop(0, nz, zero_body, 0)

    clo = e_lo // 16
    chi = (e_hi + 15) // 16
    npair = (chi - clo + 1) // 2
    headmask = jnp.where(iota < NH, _INV_SQRT_HD, 0.0)
    lanelt4 = jnp.where(iota < NH, 1.0, 0.0)
    dummy = SEG_ROWS + w
    dummyv = jnp.zeros_like(iota) + dummy
    zv = jnp.zeros((16,), jnp.float32)

    def fetch_idx(ci, ei, sem):
        pltpu.async_copy(eidx.at[pl.ds(ci * 64, 64)], ei, sem)

    def wait_idx(ei, sem):
        pltpu.make_async_copy(eidx.at[pl.ds(0, 64)], ei, sem).wait()

    def issue_gathers(ei, qb, kvb, sem):
        qi = ei[pl.ds(0, 16)]
        ki = ei[pl.ds(16, 16)]
        pltpu.async_copy(qt.at[qi], qb, sem)
        pltpu.async_copy(kvt.at[ki], kvb, sem)

    def wait_gathers(ei, qb, kvb, sem):
        pltpu.make_async_copy(qt.at[iota], qb, sem).wait()
        pltpu.make_async_copy(kvt.at[iota], kvb, sem).wait()

    def wait_scat(stg, dst, sem):
        pltpu.make_async_copy(stg, aggu_out.at[iota], sem).wait()
        pltpu.make_async_copy(dst.at[pl.ds(0, 16), pl.ds(0, DP)],
                              den_out.at[iota], sem).wait()

    def process(segv, segn, qb, kvb, stg, dst, carry):
        prevseg, den_prev, prev = carry
        inph = jnp.logical_and(segv >= s_lo, segv < s_hi)
        shifted = _perm(segv, jnp.maximum(iota - 1, 0))
        dupf = (jnp.where(jnp.logical_and(segv == shifted, iota > 0), 1.0, 0.0)
                + jnp.where(jnp.logical_and(
                    iota == 0, segv == jnp.zeros_like(segv) + prevseg),
                    1.0, 0.0))
        nextseg = jnp.where(iota < 15,
                            _perm(segv, jnp.minimum(iota + 1, 15)),
                            _perm(segn, jnp.zeros_like(iota)))
        islast = segv != nextseg
        idxsc = jnp.where(jnp.logical_and(islast, inph), segv, dummyv)
        for e in range(16):
            accs = []
            for hh in range(NH):
                acc = (qb[e, pl.ds(hh * HD, 16)]
                       * kvb[e, pl.ds(hh * HD, 16)])
                for j in range(1, 4):
                    o = hh * HD + j * 16
                    acc += qb[e, pl.ds(o, 16)] * kvb[e, pl.ds(o, 16)]
                accs.append(_allsum(acc, iota))
            s16 = jnp.where(iota == 0, accs[0],
                            jnp.where(iota == 1, accs[1],
                                      jnp.where(iota == 2, accs[2],
                                                accs[3])))
            e16 = jnp.exp(s16 * headmask) * lanelt4
            dm = _perm(dupf, jnp.full((16,), e, jnp.int32))
            den_cur = e16 + dm * den_prev
            dst[e, pl.ds(0, 16)] = den_cur
            bh = [_perm(e16, jnp.full((16,), hh, jnp.int32))
                  for hh in range(NH)]
            cur = []
            for j in range(16):
                cj = kvb[e, pl.ds(H + j * 16, 16)] * bh[j // 4] + dm * prev[j]
                stg[e, pl.ds(j * 16, 16)] = cj
                cur.append(cj)
            prev = cur
            den_prev = den_cur
        return idxsc, (segv[15], den_prev, prev)

    pltpu.sync_copy(eidx.at[pl.ds(clo * 64, 64)], eia)
    issue_gathers(eia, qba, kvba, sem_ga)
    fetch_idx(clo + 1, eib, sem_ib)
    pltpu.async_copy(stga, aggu_out.at[dummyv], sem_sa)
    pltpu.async_copy(dsta.at[pl.ds(0, 16), pl.ds(0, DP)],
                     den_out.at[dummyv], sem_sa)
    pltpu.async_copy(stgb, aggu_out.at[dummyv], sem_sb)
    pltpu.async_copy(dstb.at[pl.ds(0, 16), pl.ds(0, DP)],
                     den_out.at[dummyv], sem_sb)

    def pair_body(t, carry):
        ci = clo + 2 * t
        segva = eia[pl.ds(32, 16)]
        segna = eia[pl.ds(48, 16)]
        wait_idx(eib, sem_ib)
        segvb = eib[pl.ds(32, 16)]
        segnb = eib[pl.ds(48, 16)]
        issue_gathers(eib, qbb, kvbb, sem_gb)
        fetch_idx(ci + 2, eia, sem_ia)
        wait_scat(stga, dsta, sem_sa)
        wait_gathers(eia, qba, kvba, sem_ga)
        idxsc_a, carry = process(segva, segna, qba, kvba, stga, dsta,
                                 carry)
        pltpu.async_copy(stga, aggu_out.at[idxsc_a], sem_sa)
        pltpu.async_copy(dsta.at[pl.ds(0, 16), pl.ds(0, DP)],
                         den_out.at[idxsc_a], sem_sa)
        wait_idx(eia, sem_ia)
        issue_gathers(eia, qba, kvba, sem_ga)
        fetch_idx(ci + 3, eib, sem_ib)
        wait_scat(stgb, dstb, sem_sb)
        wait_gathers(eib, qbb, kvbb, sem_gb)
        idxsc_b, carry = process(segvb, segnb, qbb, kvbb, stgb, dstb,
                                 carry)
        pltpu.async_copy(stgb, aggu_out.at[idxsc_b], sem_sb)
        pltpu.async_copy(dstb.at[pl.ds(0, 16), pl.ds(0, DP)],
                         den_out.at[idxsc_b], sem_sb)
        return carry

    lax.fori_loop(0, npair, pair_body, (jnp.int32(-1), zv, [zv] * 16))

    wait_scat(stga, dsta, sem_sa)
    wait_scat(stgb, dstb, sem_sb)
    wait_gathers(eia, qba, kvba, sem_ga)
    wait_idx(eib, sem_ib)


def _edge_stage_sc(QT, KVT, eidx, cuts128, zb):
    mesh = plsc.VectorSubcoreMesh(core_axis_name="c", subcore_axis_name="s")
    f = pl.kernel(
        _edge_body,
        out_type=[jax.ShapeDtypeStruct((OUT_ROWS, H), jnp.float32),
                  jax.ShapeDtypeStruct((OUT_ROWS, DP), jnp.float32)],
        mesh=mesh,
        scratch_types=[
            pltpu.VMEM((16, H), jnp.float32),
            pltpu.VMEM((16, 2 * H), jnp.float32),
            pltpu.VMEM((16, H), jnp.float32),
            pltpu.VMEM((16, H), jnp.float32),
            pltpu.VMEM((64,), jnp.int32),
            pltpu.VMEM((16, H), jnp.float32),
            pltpu.VMEM((16, 2 * H), jnp.float32),
            pltpu.VMEM((16, H), jnp.float32),
            pltpu.VMEM((16, H), jnp.float32),
            pltpu.VMEM((64,), jnp.int32),
            pltpu.VMEM((128,), jnp.int32),
            pltpu.VMEM((16, H), jnp.float32),
            pltpu.SemaphoreType.DMA,
            pltpu.SemaphoreType.DMA,
            pltpu.SemaphoreType.DMA,
            pltpu.SemaphoreType.DMA,
            pltpu.SemaphoreType.DMA,
            pltpu.SemaphoreType.DMA,
        ],
    )
    aggu, den = f(QT, KVT, eidx, cuts128, zb)
    num = aggu[:N * R].reshape(N, R * H)
    den16 = den[:N * R].reshape(N, R * DP)
    return num, den16


_EXPMAT = np.zeros((R * NH, R * H), np.float32)
for _r in range(R):
    for _h in range(NH):
        _EXPMAT[_r * NH + _h, _r * H + _h * HD:_r * H + (_h + 1) * HD] = 1.0


def _post_body(h_r, num_r, den_r, expm, sew_r, rw1T, rb1, rw2T, rb2,
               wsymT, bsym, crossT, cross_b, ghT, gnT, gate_b, lng, lnb, o_r):
    hb = h_r[...]
    den2d = den_r[...]
    den4 = jnp.concatenate(
        [den2d[:, r * DP:r * DP + NH] for r in range(R)], axis=-1)
    rec4 = jnp.where(den4 > 0, 1.0 / den4, 0.0)
    recexp = _dot(rec4, expm[...])
    agg = num_r[...] * recexp
    r1 = jnp.maximum(_dot(hb, rw1T[...]) + rb1[...], 0.0)
    logits = _dot(r1, rw2T[...]) + rb2[...]
    gv = jax.nn.softmax(logits, axis=-1)
    sew = sew_r[...]
    acc = jnp.zeros((BN, H), jnp.float32)
    for k in range(K):
        sk = jnp.zeros((BN, H), jnp.float32)
        for r in range(R):
            sk = sk + agg[:, r * H:(r + 1) * H] * sew[k, r]
        tk = jnp.maximum(_dot(sk, wsymT[k]) + bsym[k][None], 0.0)
        tk = tk * gv[:, k:k + 1]
        acc = acc + _dot(tk, crossT[k])
    h_new = jnp.maximum(acc + cross_b[...], 0.0)
    g = jax.nn.sigmoid(_dot(hb, ghT[...]) + _dot(h_new, gnT[...]) + gate_b[...])
    o_r[...] = _ln(g * h_new + (1.0 - g) * hb, lng[...], lnb[...])


def _post(h, num, den16, lp, sew):
    full = lambda *s: pl.BlockSpec(s, lambda i: (0,) * len(s))
    gT = lp['gate_W'].T
    crossT = lp['cross_W'].T.reshape(K, H, H)
    return pl.pallas_call(
        _post_body,
        grid=(GRID,),
        in_specs=[
            pl.BlockSpec((BN, H), lambda i: (i, 0)),
            pl.BlockSpec((BN, R * H), lambda i: (i, 0)),
            pl.BlockSpec((BN, R * DP), lambda i: (i, 0)),
            full(R * NH, R * H),
            full(K, R),
            full(H, H), full(1, H), full(H, K), full(1, K),
            full(K, H, H), full(K, H),
            full(K, H, H), full(1, H),
            full(H, H), full(H, H), full(1, H),
            full(1, H), full(1, H),
        ],
        out_specs=pl.BlockSpec((BN, H), lambda i: (i, 0)),
        out_shape=jax.ShapeDtypeStruct((N, H), jnp.float32),
    )(h, num, den16, jnp.asarray(_EXPMAT), sew,
      lp['r_W1'].T, lp['r_b1'][None], lp['r_W2'].T, lp['r_b2'][None],
      jnp.transpose(lp['Wsym'], (0, 2, 1)), lp['bsym'],
      crossT, lp['cross_b'][None],
      gT[:H], gT[H:], lp['gate_b'][None],
      lp['ln_g'][None], lp['ln_b'][None])


def _final_body(h_r, nt_r, tattT, tatt_b, aattT, aatt_b,
                wtg1, wtg2, btg, wag1, wag2, bag,
                fo1, fo2, fo_b, fog, fobe,
                dw1T, db1, dw2T, db2, symT, sym_b, phqT, phq_b,
                dep_r, sym_r, phq_r, acc):
    i = pl.program_id(0)

    @pl.when(i == 0)
    def _():
        acc[...] = jnp.zeros_like(acc)

    hb = h_r[...]
    m0 = nt_r[...] == 0
    tl = _dot(hb, tattT[...]) + tatt_b[...]
    al = _dot(hb, aattT[...]) + aatt_b[...]
    et_ = jnp.where(m0, jnp.exp(tl), 0.0)
    ea_ = jnp.where(m0, 0.0, jnp.exp(al))
    acc[0:1, :] += jnp.sum(et_ * hb, axis=0, keepdims=True)
    acc[1:2, :] += jnp.sum(ea_ * hb, axis=0, keepdims=True)
    acc[2:3, :] += jnp.concatenate(
        [jnp.sum(et_, keepdims=True).reshape(1, 1),
         jnp.sum(ea_, keepdims=True).reshape(1, 1),
         jnp.zeros((1, H - 2), jnp.float32)], axis=-1)

    @pl.when(i == GRID - 1)
    def _():
        t_pool = acc[0:1, :] / acc[2:3, 0:1]
        a_pool = acc[1:2, :] / acc[2:3, 1:2]
        tg = jax.nn.sigmoid(_dot(t_pool, wtg1[...]) + _dot(a_pool, wtg2[...])
                            + btg[...])
        ag = jax.nn.sigmoid(_dot(a_pool, wag1[...]) + _dot(t_pool, wag2[...])
                            + bag[...])
        fused = jnp.maximum(
            _ln(_dot(tg * t_pool, fo1[...]) + _dot(ag * a_pool, fo2[...])
                + fo_b[...], fog[...], fobe[...]), 0.0)
        dep = _dot(jnp.maximum(_dot(fused, dw1T[...]) + db1[...], 0.0),
                   dw2T[...]) + db2[...]
        dep_r[...] = dep
        sym_r[...] = _dot(fused, symT[...]) + sym_b[...]
        phq_r[...] = _dot(fused, phqT[...]) + phq_b[...]


def _final(h, nt2, p):
    full = lambda *s: pl.BlockSpec(s, lambda i: (0,) * len(s))
    wtgT = p['Wtg'].T
    wagT = p['Wag'].T
    foT = p['fo_W'].T
    dep, sym, phq = pl.pallas_call(
        _final_body,
        grid=(GRID,),
        in_specs=[
            pl.BlockSpec((BN, H), lambda i: (i, 0)),
            pl.BlockSpec((BN, 1), lambda i: (i, 0)),
            full(H, 1), full(1, 1), full(H, 1), full(1, 1),
            full(H, H), full(H, H), full(1, H),
            full(H, H), full(H, H), full(1, H),
            full(H, H), full(H, H), full(1, H), full(1, H), full(1, H),
            full(H, H // 2), full(1, H // 2), full(H // 2, 1), full(1, 1),
            full(H, K), full(1, K), full(H, 1), full(1, 1),
        ],
        out_specs=[pl.BlockSpec((1, 1), lambda i: (0, 0)),
                   pl.BlockSpec((1, K), lambda i: (0, 0)),
                   pl.BlockSpec((1, 1), lambda i: (0, 0))],
        out_shape=[jax.ShapeDtypeStruct((1, 1), jnp.float32),
                   jax.ShapeDtypeStruct((1, K), jnp.float32),
                   jax.ShapeDtypeStruct((1, 1), jnp.float32)],
        scratch_shapes=[pltpu.VMEM((3, H), jnp.float32)],
    )(h, nt2,
      p['tatt_W'].T, p['tatt_b'][None], p['aatt_W'].T, p['aatt_b'][None],
      wtgT[:H], wtgT[H:], p['btg'][None],
      wagT[:H], wagT[H:], p['bag'][None],
      foT[:H], foT[H:], p['fo_b'][None], p['fo_g'][None], p['fo_be'][None],
      p['dep_W1'].T, p['dep_b1'][None], p['dep_W2'].T, p['dep_b2'][None],
      p['sym_W'].T, p['sym_b'][None], p['phq_W'].T, p['phq_b'][None])
    return dep, sym, phq


def kernel(x, pos, params, node_type, edge_index, edge_type):
    p = params
    nt2 = node_type.reshape(N, 1).astype(jnp.int32)
    src, dst = edge_index[0], edge_index[1]
    et = edge_type

    seg_full = (dst * R + et).astype(jnp.int32)
    order = jnp.argsort(seg_full)
    seg_s = seg_full[order]
    src_s = src[order].astype(jnp.int32)
    et_s = et[order].astype(jnp.int32)
    dst_s = dst[order].astype(jnp.int32)
    qidx = et_s * N + dst_s
    kidx = et_s * N + src_s
    targets = jnp.clip((jnp.arange(33) * E) // 32, 0, E - 1)
    seg_t = seg_s[targets]
    tcuts = jnp.searchsorted(seg_s, seg_t).astype(jnp.int32).at[32].set(E)
    scuts = seg_t.astype(jnp.int32).at[0].set(0).at[32].set(SEG_ROWS)
    cuts128 = jnp.concatenate(
        [tcuts[0:32], tcuts[1:33], scuts[0:32], scuts[1:33]])
    zb = jnp.zeros((16, H), jnp.float32)
    nrows = E // 16 + 4
    pad = jnp.zeros((64,), jnp.int32)
    qidx_p = jnp.concatenate([qidx, pad])
    kidx_p = jnp.concatenate([kidx, pad])
    seg_p = jnp.concatenate([seg_s, jnp.full((80,), SEG_ROWS, jnp.int32)])
    eidx = jnp.concatenate([
        qidx_p[:nrows * 16].reshape(nrows, 16),
        kidx_p[:nrows * 16].reshape(nrows, 16),
        seg_p[:nrows * 16].reshape(nrows, 16),
        seg_p[16:nrows * 16 + 16].reshape(nrows, 16),
    ], axis=1).reshape(-1)

    h = _stage_a(x, nt2, pos, p)
    for lp in p['layers']:
        Q, KV = _qkv(h, lp)
        num, den16 = _edge_stage_sc(
            Q.reshape(R * N, H), KV.reshape(R * N, 2 * H),
            eidx, cuts128, zb)
        sew = jax.nn.softmax(lp['sym_edge_logits'], axis=1)
        h = _post(h, num, den16, lp, sew)
    dep, sym, phq = _final(h, nt2, p)
    return jnp.concatenate([dep, sym, phq], axis=-1)


# speedup vs baseline: 1.0107x; 1.0107x over previous
import jax
import jax.numpy as jnp
import numpy as np
from jax import lax
from jax.experimental import pallas as pl
from jax.experimental.pallas import tpu as pltpu
from jax.experimental.pallas import tpu_sc as plsc

N = 10000
E = 160000
H = 256
R = 6
K = 8
NH = 4
HD = H // NH
BN = 1000
GRID = N // BN
_INV_SQRT_HD = 1.0 / np.sqrt(HD).astype(np.float32)


def _ln(x, g, b):
    mu = jnp.mean(x, axis=-1, keepdims=True)
    xc = x - mu
    var = jnp.mean(xc * xc, axis=-1, keepdims=True)
    return xc * jax.lax.rsqrt(var + 1e-5) * g + b


def _dot(a, b):
    return jnp.dot(a, b, preferred_element_type=jnp.float32)


def _stage_a_body(x_r, nt_r, pos_r, wtT, bt, gt, bet, waT, ba, ga, bea,
                  ntemb, w1T, b1, w2T, b2, o_r):
    xb = x_r[...]
    ht = jnp.maximum(_ln(_dot(xb, wtT[...]) + bt[...], gt[...], bet[...]), 0.0)
    ha = jnp.maximum(_ln(_dot(xb, waT[...]) + ba[...], ga[...], bea[...]), 0.0)
    m0 = nt_r[...] == 0
    h = jnp.where(m0, ht, ha)
    h = h + jnp.where(m0, ntemb[0:1, :], ntemb[1:2, :])
    pe = _dot(jnp.maximum(_dot(pos_r[...], w1T[...]) + b1[...], 0.0),
              w2T[...]) + b2[...]
    o_r[...] = h + pe


def _stage_a(x, nt2, pos, p):
    full = lambda *s: pl.BlockSpec(s, lambda i: (0,) * len(s))
    return pl.pallas_call(
        _stage_a_body,
        grid=(GRID,),
        in_specs=[
            pl.BlockSpec((BN, 777), lambda i: (i, 0)),
            pl.BlockSpec((BN, 1), lambda i: (i, 0)),
            pl.BlockSpec((BN, 1), lambda i: (i, 0)),
            full(777, H), full(1, H), full(1, H), full(1, H),
            full(777, H), full(1, H), full(1, H), full(1, H),
            full(2, H),
            full(1, H // 2), full(1, H // 2), full(H // 2, H), full(1, H),
        ],
        out_specs=pl.BlockSpec((BN, H), lambda i: (i, 0)),
        out_shape=jax.ShapeDtypeStruct((N, H), jnp.float32),
    )(x, nt2, pos,
      p['text_W'].T, p['text_b'][None], p['text_g'][None], p['text_be'][None],
      p['aud_W'].T, p['aud_b'][None], p['aud_g'][None], p['aud_be'][None],
      p['nt_emb'],
      p['pos_W1'].T, p['pos_b1'][None], p['pos_W2'].T, p['pos_b2'][None])


def _qkv_body(h_r, wqT, wkT, wvT, q_r, k_r, v_r):
    hb = h_r[...]
    for r in range(R):
        q_r[r] = _dot(hb, wqT[r])
        k_r[r] = _dot(hb, wkT[r])
        v_r[r] = _dot(hb, wvT[r])


def _qkv(h, lp):
    full3 = pl.BlockSpec((R, H, H), lambda i: (0, 0, 0))
    outspec = pl.BlockSpec((R, BN, H), lambda i: (0, i, 0))
    outshape = jax.ShapeDtypeStruct((R, N, H), jnp.float32)
    return pl.pallas_call(
        _qkv_body,
        grid=(GRID,),
        in_specs=[pl.BlockSpec((BN, H), lambda i: (i, 0)), full3, full3, full3],
        out_specs=[outspec, outspec, outspec],
        out_shape=[outshape, outshape, outshape],
    )(h, jnp.transpose(lp['Wq'], (0, 2, 1)),
      jnp.transpose(lp['Wk'], (0, 2, 1)),
      jnp.transpose(lp['Wv'], (0, 2, 1)))


SEG_ROWS = 61440
OUT_ROWS = SEG_ROWS + 32
DP = 128


_GDN = lax.GatherDimensionNumbers(offset_dims=(), collapsed_slice_dims=(0,),
                                  start_index_map=(0,))


def _perm(v, idx):
    return lax.gather(v, idx[:, None], _GDN, (1,),
                      mode=lax.GatherScatterMode.PROMISE_IN_BOUNDS)


def _allsum(v, iota):
    for step in (1, 2, 4, 8):
        v = v + _perm(v, jnp.bitwise_xor(iota, step))
    return v


def _edge_body(qt, kt, vt, qidx, kidx, seg, cuts, zb,
               aggu_out, den_out,
               qba, kba, vba, stga, dsta, qia, kia, sga,
               qbb, kbb, vbb, stgb, dstb, qib, kib, sgb,
               ct_v, zrows,
               sem_ga, sem_gb, sem_ia, sem_ib, sem_sa, sem_sb):
    c = lax.axis_index("c")
    s = lax.axis_index("s")
    w = c * 16 + s
    iota = lax.iota(jnp.int32, 16)
    pltpu.sync_copy(zb, zrows)
    pltpu.sync_copy(cuts, ct_v)
    for dd in (dsta, dstb):
        for e in range(16):
            for j in range(1, DP // 16):
                dd[e, pl.ds(j * 16, 16)] = jnp.zeros((16,), jnp.float32)

    def ext(off):
        a0 = ct_v[pl.ds(off, 16)]
        a1 = ct_v[pl.ds(off + 16, 16)]
        ga = jnp.clip(w, 0, 15)
        gb = jnp.clip(w - 16, 0, 15)
        av = _perm(a0, jnp.where(iota == 0, jnp.zeros_like(iota) + ga, iota))[0]
        bv = _perm(a1, jnp.where(iota == 0, jnp.zeros_like(iota) + gb, iota))[0]
        return jnp.where(w < 16, av, bv)

    e_lo = ext(0)
    e_hi = ext(32)
    s_lo = ext(64)
    s_hi = ext(96)

    nz = (s_hi - s_lo + 15) // 16

    def zero_body(j, carry):
        idxz = jnp.minimum(s_lo + j * 16 + iota, s_hi - 1)
        z1 = pltpu.async_copy(zrows, aggu_out.at[idxz], sem_sa)
        z2 = pltpu.async_copy(zrows.at[pl.ds(0, 16), pl.ds(0, DP)],
                              den_out.at[idxz], sem_sa)
        z1.wait()
        z2.wait()
        return carry

    lax.fori_loop(0, nz, zero_body, 0)

    clo = e_lo // 16
    chi = (e_hi + 15) // 16
    npair = (chi - clo + 1) // 2
    headmask = jnp.where(iota < NH, _INV_SQRT_HD, 0.0)
    lanelt4 = jnp.where(iota < NH, 1.0, 0.0)
    dummy = SEG_ROWS + w
    dummyv = jnp.zeros_like(iota) + dummy
    zv = jnp.zeros((16,), jnp.float32)

    def fetch_idx(base, qi, ki, sg, sem):
        d1 = pltpu.async_copy(qidx.at[pl.ds(base, 16)], qi, sem)
        d2 = pltpu.async_copy(kidx.at[pl.ds(base, 16)], ki, sem)
        d3 = pltpu.async_copy(seg.at[pl.ds(base, 32)], sg, sem)
        return d1, d2, d3

    def wait_idx(qi, ki, sg, sem):
        pltpu.make_async_copy(qidx.at[pl.ds(0, 16)], qi, sem).wait()
        pltpu.make_async_copy(kidx.at[pl.ds(0, 16)], ki, sem).wait()
        pltpu.make_async_copy(seg.at[pl.ds(0, 32)], sg, sem).wait()

    def issue_gathers(qi, ki, qb, kb, vb, sem):
        pltpu.async_copy(qt.at[qi], qb, sem)
        pltpu.async_copy(kt.at[ki], kb, sem)
        pltpu.async_copy(vt.at[ki], vb, sem)

    def wait_gathers(qi, ki, qb, kb, vb, sem):
        pltpu.make_async_copy(qt.at[qi], qb, sem).wait()
        pltpu.make_async_copy(kt.at[ki], kb, sem).wait()
        pltpu.make_async_copy(vt.at[ki], vb, sem).wait()

    def wait_scat(stg, dst, sem):
        pltpu.make_async_copy(stg, aggu_out.at[iota], sem).wait()
        pltpu.make_async_copy(dst.at[pl.ds(0, 16), pl.ds(0, DP)],
                              den_out.at[iota], sem).wait()

    def process(segv, segn, qb, kb, vb, stg, dst, carry):
        prevseg, den_prev, prev = carry
        inph = jnp.logical_and(segv >= s_lo, segv < s_hi)
        shifted = _perm(segv, jnp.maximum(iota - 1, 0))
        dupf = (jnp.where(jnp.logical_and(segv == shifted, iota > 0), 1.0, 0.0)
                + jnp.where(jnp.logical_and(
                    iota == 0, segv == jnp.zeros_like(segv) + prevseg),
                    1.0, 0.0))
        nextseg = jnp.where(iota < 15,
                            _perm(segv, jnp.minimum(iota + 1, 15)),
                            _perm(segn, jnp.zeros_like(iota)))
        islast = segv != nextseg
        idxsc = jnp.where(jnp.logical_and(islast, inph), segv, dummyv)
        for e in range(16):
            accs = []
            for hh in range(NH):
                acc = (qb[e, pl.ds(hh * HD, 16)]
                       * kb[e, pl.ds(hh * HD, 16)])
                for j in range(1, 4):
                    o = hh * HD + j * 16
                    acc += qb[e, pl.ds(o, 16)] * kb[e, pl.ds(o, 16)]
                accs.append(_allsum(acc, iota))
            s16 = jnp.where(iota == 0, accs[0],
                            jnp.where(iota == 1, accs[1],
                                      jnp.where(iota == 2, accs[2],
                                                accs[3])))
            e16 = jnp.exp(s16 * headmask) * lanelt4
            dm = _perm(dupf, jnp.full((16,), e, jnp.int32))
            den_cur = e16 + dm * den_prev
            dst[e, pl.ds(0, 16)] = den_cur
            bh = [_perm(e16, jnp.full((16,), hh, jnp.int32))
                  for hh in range(NH)]
            cur = []
            for j in range(16):
                cj = vb[e, pl.ds(j * 16, 16)] * bh[j // 4] + dm * prev[j]
                stg[e, pl.ds(j * 16, 16)] = cj
                cur.append(cj)
            prev = cur
            den_prev = den_cur
        return idxsc, (segv[15], den_prev, prev)

    pltpu.sync_copy(qidx.at[pl.ds(clo * 16, 16)], qia)
    pltpu.sync_copy(kidx.at[pl.ds(clo * 16, 16)], kia)
    pltpu.sync_copy(seg.at[pl.ds(clo * 16, 32)], sga)
    issue_gathers(qia, kia, qba, kba, vba, sem_ga)
    fetch_idx((clo + 1) * 16, qib, kib, sgb, sem_ib)
    pltpu.async_copy(stga, aggu_out.at[dummyv], sem_sa)
    pltpu.async_copy(dsta.at[pl.ds(0, 16), pl.ds(0, DP)],
                     den_out.at[dummyv], sem_sa)
    pltpu.async_copy(stgb, aggu_out.at[dummyv], sem_sb)
    pltpu.async_copy(dstb.at[pl.ds(0, 16), pl.ds(0, DP)],
                     den_out.at[dummyv], sem_sb)

    def pair_body(t, carry):
        ci = clo + 2 * t
        segva = sga[pl.ds(0, 16)]
        segna = sga[pl.ds(16, 16)]
        wait_idx(qib, kib, sgb, sem_ib)
        segvb = sgb[pl.ds(0, 16)]
        segnb = sgb[pl.ds(16, 16)]
        issue_gathers(qib, kib, qbb, kbb, vbb, sem_gb)
        fetch_idx((ci + 2) * 16, qia, kia, sga, sem_ia)
        wait_scat(stga, dsta, sem_sa)
        wait_gathers(qia, kia, qba, kba, vba, sem_ga)
        idxsc_a, carry = process(segva, segna, qba, kba, vba, stga, dsta,
                                 carry)
        pltpu.async_copy(stga, aggu_out.at[idxsc_a], sem_sa)
        pltpu.async_copy(dsta.at[pl.ds(0, 16), pl.ds(0, DP)],
                         den_out.at[idxsc_a], sem_sa)
        wait_idx(qia, kia, sga, sem_ia)
        issue_gathers(qia, kia, qba, kba, vba, sem_ga)
        fetch_idx((ci + 3) * 16, qib, kib, sgb, sem_ib)
        wait_scat(stgb, dstb, sem_sb)
        wait_gathers(qib, kib, qbb, kbb, vbb, sem_gb)
        idxsc_b, carry = process(segvb, segnb, qbb, kbb, vbb, stgb, dstb,
                                 carry)
        pltpu.async_copy(stgb, aggu_out.at[idxsc_b], sem_sb)
        pltpu.async_copy(dstb.at[pl.ds(0, 16), pl.ds(0, DP)],
                         den_out.at[idxsc_b], sem_sb)
        return carry

    lax.fori_loop(0, npair, pair_body, (jnp.int32(-1), zv, [zv] * 16))

    wait_scat(stga, dsta, sem_sa)
    wait_scat(stgb, dstb, sem_sb)
    wait_gathers(qia, kia, qba, kba, vba, sem_ga)
    wait_idx(qib, kib, sgb, sem_ib)


def _edge_stage_sc(QT, KT, VT, qidx, kidx, seg_s, cuts128, zb):
    mesh = plsc.VectorSubcoreMesh(core_axis_name="c", subcore_axis_name="s")
    f = pl.kernel(
        _edge_body,
        out_type=[jax.ShapeDtypeStruct((OUT_ROWS, H), jnp.float32),
                  jax.ShapeDtypeStruct((OUT_ROWS, DP), jnp.float32)],
        mesh=mesh,
        scratch_types=[
            pltpu.VMEM((16, H), jnp.float32),
            pltpu.VMEM((16, H), jnp.float32),
            pltpu.VMEM((16, H), jnp.float32),
            pltpu.VMEM((16, H), jnp.float32),
            pltpu.VMEM((16, H), jnp.float32),
            pltpu.VMEM((16,), jnp.int32),
            pltpu.VMEM((16,), jnp.int32),
            pltpu.VMEM((32,), jnp.int32),
            pltpu.VMEM((16, H), jnp.float32),
            pltpu.VMEM((16, H), jnp.float32),
            pltpu.VMEM((16, H), jnp.float32),
            pltpu.VMEM((16, H), jnp.float32),
            pltpu.VMEM((16, H), jnp.float32),
            pltpu.VMEM((16,), jnp.int32),
            pltpu.VMEM((16,), jnp.int32),
            pltpu.VMEM((32,), jnp.int32),
            pltpu.VMEM((128,), jnp.int32),
            pltpu.VMEM((16, H), jnp.float32),
            pltpu.SemaphoreType.DMA,
            pltpu.SemaphoreType.DMA,
            pltpu.SemaphoreType.DMA,
            pltpu.SemaphoreType.DMA,
            pltpu.SemaphoreType.DMA,
            pltpu.SemaphoreType.DMA,
        ],
    )
    pad = jnp.zeros((64,), jnp.int32)
    qidx_p = jnp.concatenate([qidx, pad])
    kidx_p = jnp.concatenate([kidx, pad])
    seg_p = jnp.concatenate([seg_s, jnp.full((64,), SEG_ROWS, jnp.int32)])
    aggu, den = f(QT, KT, VT, qidx_p, kidx_p, seg_p, cuts128, zb)
    num = aggu[:N * R].reshape(N, R * H)
    den16 = den[:N * R].reshape(N, R * DP)
    return num, den16


_EXPMAT = np.zeros((R * NH, R * H), np.float32)
for _r in range(R):
    for _h in range(NH):
        _EXPMAT[_r * NH + _h, _r * H + _h * HD:_r * H + (_h + 1) * HD] = 1.0


def _post_body(h_r, num_r, den_r, expm, sew_r, rw1T, rb1, rw2T, rb2,
               wsymT, bsym, crossT, cross_b, ghT, gnT, gate_b, lng, lnb, o_r):
    hb = h_r[...]
    den2d = den_r[...]
    den4 = jnp.concatenate(
        [den2d[:, r * DP:r * DP + NH] for r in range(R)], axis=-1)
    rec4 = jnp.where(den4 > 0, 1.0 / den4, 0.0)
    recexp = _dot(rec4, expm[...])
    agg = num_r[...] * recexp
    r1 = jnp.maximum(_dot(hb, rw1T[...]) + rb1[...], 0.0)
    logits = _dot(r1, rw2T[...]) + rb2[...]
    gv = jax.nn.softmax(logits, axis=-1)
    sew = sew_r[...]
    acc = jnp.zeros((BN, H), jnp.float32)
    for k in range(K):
        sk = jnp.zeros((BN, H), jnp.float32)
        for r in range(R):
            sk = sk + agg[:, r * H:(r + 1) * H] * sew[k, r]
        tk = jnp.maximum(_dot(sk, wsymT[k]) + bsym[k][None], 0.0)
        tk = tk * gv[:, k:k + 1]
        acc = acc + _dot(tk, crossT[k])
    h_new = jnp.maximum(acc + cross_b[...], 0.0)
    g = jax.nn.sigmoid(_dot(hb, ghT[...]) + _dot(h_new, gnT[...]) + gate_b[...])
    o_r[...] = _ln(g * h_new + (1.0 - g) * hb, lng[...], lnb[...])


def _post(h, num, den16, lp, sew):
    full = lambda *s: pl.BlockSpec(s, lambda i: (0,) * len(s))
    gT = lp['gate_W'].T
    crossT = lp['cross_W'].T.reshape(K, H, H)
    return pl.pallas_call(
        _post_body,
        grid=(GRID,),
        in_specs=[
            pl.BlockSpec((BN, H), lambda i: (i, 0)),
            pl.BlockSpec((BN, R * H), lambda i: (i, 0)),
            pl.BlockSpec((BN, R * DP), lambda i: (i, 0)),
            full(R * NH, R * H),
            full(K, R),
            full(H, H), full(1, H), full(H, K), full(1, K),
            full(K, H, H), full(K, H),
            full(K, H, H), full(1, H),
            full(H, H), full(H, H), full(1, H),
            full(1, H), full(1, H),
        ],
        out_specs=pl.BlockSpec((BN, H), lambda i: (i, 0)),
        out_shape=jax.ShapeDtypeStruct((N, H), jnp.float32),
    )(h, num, den16, jnp.asarray(_EXPMAT), sew,
      lp['r_W1'].T, lp['r_b1'][None], lp['r_W2'].T, lp['r_b2'][None],
      jnp.transpose(lp['Wsym'], (0, 2, 1)), lp['bsym'],
      crossT, lp['cross_b'][None],
      gT[:H], gT[H:], lp['gate_b'][None],
      lp['ln_g'][None], lp['ln_b'][None])


def _final_body(h_r, nt_r, tattT, tatt_b, aattT, aatt_b,
                wtg1, wtg2, btg, wag1, wag2, bag,
                fo1, fo2, fo_b, fog, fobe,
                dw1T, db1, dw2T, db2, symT, sym_b, phqT, phq_b,
                dep_r, sym_r, phq_r, acc):
    i = pl.program_id(0)

    @pl.when(i == 0)
    def _():
        acc[...] = jnp.zeros_like(acc)

    hb = h_r[...]
    m0 = nt_r[...] == 0
    tl = _dot(hb, tattT[...]) + tatt_b[...]
    al = _dot(hb, aattT[...]) + aatt_b[...]
    et_ = jnp.where(m0, jnp.exp(tl), 0.0)
    ea_ = jnp.where(m0, 0.0, jnp.exp(al))
    acc[0:1, :] += jnp.sum(et_ * hb, axis=0, keepdims=True)
    acc[1:2, :] += jnp.sum(ea_ * hb, axis=0, keepdims=True)
    acc[2:3, :] += jnp.concatenate(
        [jnp.sum(et_, keepdims=True).reshape(1, 1),
         jnp.sum(ea_, keepdims=True).reshape(1, 1),
         jnp.zeros((1, H - 2), jnp.float32)], axis=-1)

    @pl.when(i == GRID - 1)
    def _():
        t_pool = acc[0:1, :] / acc[2:3, 0:1]
        a_pool = acc[1:2, :] / acc[2:3, 1:2]
        tg = jax.nn.sigmoid(_dot(t_pool, wtg1[...]) + _dot(a_pool, wtg2[...])
                            + btg[...])
        ag = jax.nn.sigmoid(_dot(a_pool, wag1[...]) + _dot(t_pool, wag2[...])
                            + bag[...])
        fused = jnp.maximum(
            _ln(_dot(tg * t_pool, fo1[...]) + _dot(ag * a_pool, fo2[...])
                + fo_b[...], fog[...], fobe[...]), 0.0)
        dep = _dot(jnp.maximum(_dot(fused, dw1T[...]) + db1[...], 0.0),
                   dw2T[...]) + db2[...]
        dep_r[...] = dep
        sym_r[...] = _dot(fused, symT[...]) + sym_b[...]
        phq_r[...] = _dot(fused, phqT[...]) + phq_b[...]


def _final(h, nt2, p):
    full = lambda *s: pl.BlockSpec(s, lambda i: (0,) * len(s))
    wtgT = p['Wtg'].T
    wagT = p['Wag'].T
    foT = p['fo_W'].T
    dep, sym, phq = pl.pallas_call(
        _final_body,
        grid=(GRID,),
        in_specs=[
            pl.BlockSpec((BN, H), lambda i: (i, 0)),
            pl.BlockSpec((BN, 1), lambda i: (i, 0)),
            full(H, 1), full(1, 1), full(H, 1), full(1, 1),
            full(H, H), full(H, H), full(1, H),
            full(H, H), full(H, H), full(1, H),
            full(H, H), full(H, H), full(1, H), full(1, H), full(1, H),
            full(H, H // 2), full(1, H // 2), full(H // 2, 1), full(1, 1),
            full(H, K), full(1, K), full(H, 1), full(1, 1),
        ],
        out_specs=[pl.BlockSpec((1, 1), lambda i: (0, 0)),
                   pl.BlockSpec((1, K), lambda i: (0, 0)),
                   pl.BlockSpec((1, 1), lambda i: (0, 0))],
        out_shape=[jax.ShapeDtypeStruct((1, 1), jnp.float32),
                   jax.ShapeDtypeStruct((1, K), jnp.float32),
                   jax.ShapeDtypeStruct((1, 1), jnp.float32)],
        scratch_shapes=[pltpu.VMEM((3, H), jnp.float32)],
    )(h, nt2,
      p['tatt_W'].T, p['tatt_b'][None], p['aatt_W'].T, p['aatt_b'][None],
      wtgT[:H], wtgT[H:], p['btg'][None],
      wagT[:H], wagT[H:], p['bag'][None],
      foT[:H], foT[H:], p['fo_b'][None], p['fo_g'][None], p['fo_be'][None],
      p['dep_W1'].T, p['dep_b1'][None], p['dep_W2'].T, p['dep_b2'][None],
      p['sym_W'].T, p['sym_b'][None], p['phq_W'].T, p['phq_b'][None])
    return dep, sym, phq


def kernel(x, pos, params, node_type, edge_index, edge_type):
    p = params
    nt2 = node_type.reshape(N, 1).astype(jnp.int32)
    src, dst = edge_index[0], edge_index[1]
    et = edge_type

    seg_full = (dst * R + et).astype(jnp.int32)
    order = jnp.argsort(seg_full)
    seg_s = seg_full[order]
    src_s = src[order].astype(jnp.int32)
    et_s = et[order].astype(jnp.int32)
    dst_s = dst[order].astype(jnp.int32)
    qidx = et_s * N + dst_s
    kidx = et_s * N + src_s
    targets = jnp.clip((jnp.arange(33) * E) // 32, 0, E - 1)
    seg_t = seg_s[targets]
    tcuts = jnp.searchsorted(seg_s, seg_t).astype(jnp.int32).at[32].set(E)
    scuts = seg_t.astype(jnp.int32).at[0].set(0).at[32].set(SEG_ROWS)
    cuts128 = jnp.concatenate(
        [tcuts[0:32], tcuts[1:33], scuts[0:32], scuts[1:33]])
    zb = jnp.zeros((16, H), jnp.float32)

    h = _stage_a(x, nt2, pos, p)
    for lp in p['layers']:
        Q, Kx, V = _qkv(h, lp)
        num, den16 = _edge_stage_sc(
            Q.reshape(R * N, H), Kx.reshape(R * N, H), V.reshape(R * N, H),
            qidx, kidx, seg_s, cuts128, zb)
        sew = jax.nn.softmax(lp['sym_edge_logits'], axis=1)
        h = _post(h, num, den16, lp, sew)
    dep, sym, phq = _final(h, nt2, p)
    return jnp.concatenate([dep, sym, phq], axis=-1)
